# baseline (speedup 1.0000x reference)
import functools

import jax
import jax.numpy as jnp
from jax.experimental import pallas as pl
from jax.experimental.pallas import tpu as pltpu

B_, N1, IN_C, HID, HEADS, OUT_C, DEG = 256, 90, 90, 128, 8, 2, 16
D = HID // HEADS
P1, P2, P3 = 72, 57, 45
F32 = jnp.float32
ETS = [('bb', 'b', 'b'), ('dd', 'd', 'd'), ('bd', 'b', 'd'), ('db', 'd', 'b')]
SCORE_ETS = ('bb', 'db', 'dd', 'bd')


def _dot(a, b):
    return jnp.dot(a, b, preferred_element_type=F32)


def _dotf(a, b):
    return jnp.dot(a, b, preferred_element_type=F32,
                   precision=jax.lax.Precision.DEFAULT)


def _dgT(a, b):
    return jax.lax.dot_general(a, b, (((0,), (0,)), ((), ())),
                               preferred_element_type=F32)


def _softmax_rows(x):
    m = jnp.max(x, axis=-1, keepdims=True)
    e = jnp.exp(x - m)
    return e / jnp.sum(e, axis=-1, keepdims=True)


def _build_A(srcT, n):
    nw = (n + 31) // 32
    val = jnp.left_shift(jnp.int32(1), jnp.bitwise_and(srcT, 31))
    word = jnp.right_shift(srcT, 5)
    rows = []
    for w in range(nw):
        m = jnp.where(word == w, val, 0)
        r = m[0:8] | m[8:16]
        r = r[0:4] | r[4:8]
        r = r[0:2] | r[2:4]
        rows.append(r[0:1] | r[1:2])
    W = jnp.concatenate(rows, axis=0).T
    bits = jax.lax.broadcasted_iota(jnp.int32, (n, 32), 1)
    segs = [jnp.bitwise_and(jnp.right_shift(W[:, w:w + 1], bits), 1)
            for w in range(nw)]
    return jnp.concatenate(segs, axis=1)[:, :n].astype(F32)


def _attention_et(hs, hd, A, sel8, asrcT, adstT):
    n = hs.shape[0]
    a_s = _dot(hs, asrcT)
    a_d = _dot(hd, adstT)
    c = a_d + jnp.max(a_s, axis=0, keepdims=True)
    c = jnp.maximum(c, 0.2 * c)
    a_sP = jnp.concatenate([a_s.T, jnp.zeros((HEADS, 128 - n), F32)], axis=1)
    y = sel8 * jnp.concatenate([a_sP] * HEADS, axis=1)
    lhs = jnp.concatenate([a_d, jnp.ones((n, HEADS), F32), c], axis=1)
    u2 = _dotf(lhs, jnp.concatenate([sel8, y, -sel8], axis=0))
    u3 = _dotf(lhs, jnp.concatenate([0.2 * sel8, 0.2 * y, -sel8], axis=0))
    a128 = jnp.concatenate([A + 1e-9, jnp.zeros((n, 128 - n), F32)], axis=1)
    e_all = jnp.exp(jnp.maximum(u2, u3)) \
        * jnp.concatenate([a128] * HEADS, axis=1)
    ones = jnp.ones((n, 1), F32)
    zpad = jnp.zeros((128 - n, D + 1), F32)
    cols = []
    for hh in range(HEADS):
        e_h = e_all[:, 128 * hh:128 * (hh + 1)]
        hp = jnp.concatenate(
            [jnp.concatenate([hs[:, hh * D:(hh + 1) * D], ones], axis=1),
             zpad], axis=0)
        oz = _dotf(e_h, hp)
        cols.append(oz[:, 0:D] / oz[:, D:D + 1])
    return jnp.maximum(jnp.concatenate(cols, axis=1), 0.0)


def _conv_a(bb, xb3, xd3, Amap, sel8, Wb, Wd, asrcT, adstT, kW, kb, q,
            o_refs, sc_ref, n):
    h2 = {'b': _dot(xb3.reshape(bb * n, -1), Wb),
          'd': _dot(xd3.reshape(bb * n, -1), Wd)}
    h = {nt: [h2[nt][g * n:(g + 1) * n] for g in range(bb)] for nt in h2}
    o = {}
    for i, (et, s, d) in enumerate(ETS):
        o[et] = [_attention_et(h[s][g], h[d][g], Amap[et][g], sel8,
                               asrcT[et], adstT[et]) for g in range(bb)]
        for g in range(bb):
            o_refs[i][g] = o[et][g]
    big = jnp.concatenate([o[et][g] for et in SCORE_ETS for g in range(bb)],
                          axis=0)
    t = jnp.tanh(_dot(big, kW) + kb) * q
    sums = jnp.sum(t.reshape(4, bb * n, HID), axis=1)
    contrib = jnp.concatenate([sums, jnp.zeros((4, HID), F32)], axis=0)
    b = pl.program_id(0)
    prev = jnp.where(b == 0, jnp.zeros((8, 128), F32), sc_ref[...])
    sc_ref[...] = prev + contrib
    return o


def _combine3(w_ref, o_bb, o_dd, o_bd, o_db):
    xb = jnp.maximum(w_ref[0:1, 0:1] * o_bb[...] + w_ref[1:2, 0:1] * o_db[...], 0.0)
    xd = jnp.maximum(w_ref[2:3, 0:1] * o_dd[...] + w_ref[3:4, 0:1] * o_bd[...], 0.0)
    return xb, xd


def _readout(xnb, xnd):
    parts = [jnp.max(xnb, axis=0, keepdims=True),
             jnp.mean(xnb, axis=0, keepdims=True),
             jnp.max(xnd, axis=0, keepdims=True),
             jnp.mean(xnd, axis=0, keepdims=True)]
    return jnp.concatenate(parts, axis=1)


def _pool(bb, n, p_dim, xb3, xd3, Wp_b, Wp_d, A, Pb_o, Pd_o, xr_o):
    Sb2 = _softmax_rows(_dot(xb3.reshape(bb * n, HID), Wp_b))
    Sd2 = _softmax_rows(_dot(xd3.reshape(bb * n, HID), Wp_d))
    Pb_o[...] = Sb2.reshape(bb, n, p_dim)
    Pd_o[...] = Sd2.reshape(bb, n, p_dim)
    S = {'b': [Sb2[g * n:(g + 1) * n] for g in range(bb)],
         'd': [Sd2[g * n:(g + 1) * n] for g in range(bb)]}
    xnb, xnd, An = [], [], {et: [] for et, _, _ in ETS}
    for g in range(bb):
        St = {'b': S['b'][g].T, 'd': S['d'][g].T}
        xnb.append(_dot(St['b'], xb3[g]))
        xnd.append(_dot(St['d'], xd3[g]))
        xr_o[g] = _readout(xnb[-1], xnd[-1])
        for et, s, d in ETS:
            An[et].append(_dot(St[d], _dot(A[et][g], S[s][g])))
    return xnb, xnd, An


def _k1a_body(bb,
              xb_ref, xd_ref, s_bb, s_dd, s_bd, s_db, sel8,
              Wb, Wd, as_bb, as_dd, as_bd, as_db, ad_bb, ad_dd, ad_bd, ad_db,
              kW, kb, q,
              o_bb, o_dd, o_bd, o_db, A_bb, A_dd, A_bd, A_db, sc_ref):
    b = pl.program_id(0)
    src = {'bb': s_bb, 'dd': s_dd, 'bd': s_bd, 'db': s_db}
    Amap = {et: [_build_A(src[et][g] - (b * bb + g) * N1, N1)
                 for g in range(bb)] for et, _, _ in ETS}
    A_o = {'bb': A_bb, 'dd': A_dd, 'bd': A_bd, 'db': A_db}
    for et, _, _ in ETS:
        for g in range(bb):
            A_o[et][g] = Amap[et][g]
    asrcT = {'bb': as_bb[...], 'dd': as_dd[...], 'bd': as_bd[...], 'db': as_db[...]}
    adstT = {'bb': ad_bb[...], 'dd': ad_dd[...], 'bd': ad_bd[...], 'db': ad_db[...]}
    _conv_a(bb, xb_ref[...], xd_ref[...], Amap, sel8[...], Wb[...], Wd[...],
            asrcT, adstT,
            kW[...], kb[...], q[...], [o_bb, o_dd, o_bd, o_db], sc_ref, N1)


def _stage_body(bb, n, p_dim,
                o1_bb, o1_dd, o1_bd, o1_db, a_bb, a_dd, a_bd, a_db, w_ref,
                sel8, Wp_b, Wp_d, Wb, Wd,
                as_bb, as_dd, as_bd, as_db, ad_bb, ad_dd, ad_bd, ad_db,
                kW, kb, q,
                Pb_o, Pd_o, xr_o, An_bb, An_dd, An_bd, An_db,
                o2_bb, o2_dd, o2_bd, o2_db, sc_ref):
    a_in = {'bb': a_bb, 'dd': a_dd, 'bd': a_bd, 'db': a_db}
    A = {et: [a_in[et][g] for g in range(bb)] for et, _, _ in ETS}
    xb3, xd3 = _combine3(w_ref, o1_bb, o1_dd, o1_bd, o1_db)
    xnb, xnd, An = _pool(bb, n, p_dim, xb3, xd3, Wp_b[...], Wp_d[...], A,
                         Pb_o, Pd_o, xr_o)
    An_o = {'bb': An_bb, 'dd': An_dd, 'bd': An_bd, 'db': An_db}
    for et, _, _ in ETS:
        for g in range(bb):
            An_o[et][g] = An[et][g]
    asrcT = {'bb': as_bb[...], 'dd': as_dd[...], 'bd': as_bd[...], 'db': as_db[...]}
    adstT = {'bb': ad_bb[...], 'dd': ad_dd[...], 'bd': ad_bd[...], 'db': ad_db[...]}
    xnb3 = jnp.concatenate([x[None] for x in xnb], axis=0)
    xnd3 = jnp.concatenate([x[None] for x in xnd], axis=0)
    _conv_a(bb, xnb3, xnd3, An, sel8[...], Wb[...], Wd[...], asrcT, adstT,
            kW[...], kb[...], q[...], [o2_bb, o2_dd, o2_bd, o2_db], sc_ref,
            p_dim)


def _k3b_body(bb, o_bb, o_dd, o_bd, o_db, w_ref, Wp_b, Wp_d, Pb_o, Pd_o, xr_o):
    n, p_dim = P2, P3
    xb3, xd3 = _combine3(w_ref, o_bb, o_dd, o_bd, o_db)
    Sb2 = _softmax_rows(_dot(xb3.reshape(bb * n, HID), Wp_b[...]))
    Sd2 = _softmax_rows(_dot(xd3.reshape(bb * n, HID), Wp_d[...]))
    Pb_o[...] = Sb2.reshape(bb, n, p_dim)
    Pd_o[...] = Sd2.reshape(bb, n, p_dim)
    for g in range(bb):
        xnb = _dot(Sb2[g * n:(g + 1) * n].T, xb3[g])
        xnd = _dot(Sd2[g * n:(g + 1) * n].T, xd3[g])
        xr_o[g] = _readout(xnb, xnd)


def _k4_body(x1_ref, x2_ref, x3_ref, l1W, l1b, l2W, l2b, l3W, l3b,
             out_o, h_o):
    nb = x1_ref.shape[0]
    s = (x1_ref[...] + x2_ref[...] + x3_ref[...]).reshape(nb, 4 * HID)
    s = s - jnp.mean(s, axis=0, keepdims=True)
    rn = jnp.sqrt(1e-6 + jnp.sum(s * s, axis=1, keepdims=True))
    feat = 100.0 * s / rn
    h1 = jnp.maximum(_dot(feat, l1W[...]) + l1b[...], 0.0)
    h2 = jnp.maximum(_dot(h1, l2W[...]) + l2b[...], 0.0)
    out_o[...] = _dot(h2, l3W[...]) + l3b[...]
    h_o[...] = h2


def _att_mat(a):
    m = jnp.zeros((HID, HEADS), F32)
    return m.at[jnp.arange(HID), jnp.arange(HID) // D].set(a.reshape(HID))


def _w_pack(score_out, nb, n):
    s = jnp.sum(score_out, axis=1)[0:4] / (nb * n)
    w = jnp.concatenate([jax.nn.softmax(s[0:2]), jax.nn.softmax(s[2:4])])
    w = jnp.concatenate([w, jnp.zeros((4,), F32)])
    return jnp.broadcast_to(w[:, None], (8, 128))


def _full(shape):
    nd = len(shape)
    return pl.BlockSpec(shape, lambda b, _nd=nd: (0,) * _nd)


def _perg(bb, shape):
    nd = len(shape)
    return pl.BlockSpec((bb,) + shape, lambda b, _nd=nd: (b,) + (0,) * _nd)


def _cparams():
    return pltpu.CompilerParams(dimension_semantics=("arbitrary",))


def kernel(x_bold, x_dti, ei_bb, ei_dd, ei_bd, ei_db, params):
    nb = x_bold.shape[0] // N1
    bb = 4 if nb % 4 == 0 else 1
    f32 = lambda shape: jax.ShapeDtypeStruct(shape, F32)

    srcs = [ei[0].astype(jnp.int32).reshape(nb, N1, DEG).transpose(0, 2, 1)
            for ei in (ei_bb, ei_dd, ei_bd, ei_db)]
    xb = x_bold.reshape(nb, N1, IN_C)
    xd = x_dti.reshape(nb, N1, IN_C)

    convs = [params['conv%d' % (l + 1)] for l in range(3)]
    pools = [params['pool%d' % (l + 1)] for l in range(3)]
    att = []
    for c in convs:
        att.append(([_att_mat(c['att_src'][et]) for et, _, _ in ETS],
                    [_att_mat(c['att_dst'][et]) for et, _, _ in ETS]))
    kWs = [c['k_W'] for c in convs]
    kbs = [c['k_b'].reshape(1, HID) for c in convs]
    qs = [c['q'].reshape(1, HID) for c in convs]
    r8 = jnp.arange(HEADS)[:, None]
    sel8 = (jnp.arange(HEADS * 128)[None, :] // 128 == r8).astype(F32)

    c = convs[0]
    o1 = pl.pallas_call(
        functools.partial(_k1a_body, bb),
        grid=(nb // bb,),
        in_specs=[_perg(bb, (N1, IN_C))] * 2 + [_perg(bb, (DEG, N1))] * 4
                 + [_full((HEADS, 1024))]
                 + [_full((IN_C, HID))] * 2 + [_full((HID, HEADS))] * 8
                 + [_full((HID, HID)), _full((1, HID)), _full((1, HID))],
        out_specs=[_perg(bb, (N1, HID))] * 4 + [_perg(bb, (N1, N1))] * 4
                  + [_full((8, 128))],
        out_shape=[f32((nb, N1, HID))] * 4 + [f32((nb, N1, N1))] * 4
                  + [f32((8, 128))],
        compiler_params=_cparams(),
    )(xb, xd, *srcs, sel8, c['W']['bold'], c['W']['dti'],
      *att[0][0], *att[0][1], kWs[0], kbs[0], qs[0])
    w1 = _w_pack(o1[8], nb, N1)

    def stage(l, n, p_dim, w, o_prev, a_args, a_specs):
        cn = convs[l]
        body = functools.partial(_stage_body, bb, n, p_dim)
        return pl.pallas_call(
            body,
            grid=(nb // bb,),
            in_specs=[_perg(bb, (n, HID))] * 4 + a_specs + [_full((8, 128))]
                     + [_full((HEADS, 1024))]
                     + [_full((HID, p_dim))] * 2 + [_full((HID, HID))] * 2
                     + [_full((HID, HEADS))] * 8
                     + [_full((HID, HID)), _full((1, HID)), _full((1, HID))],
            out_specs=[_perg(bb, (n, p_dim))] * 2 + [_perg(bb, (1, 4 * HID))]
                      + [_perg(bb, (p_dim, p_dim))] * 4
                      + [_perg(bb, (p_dim, HID))] * 4 + [_full((8, 128))],
            out_shape=[f32((nb, n, p_dim))] * 2 + [f32((nb, 1, 4 * HID))]
                      + [f32((nb, p_dim, p_dim))] * 4
                      + [f32((nb, p_dim, HID))] * 4 + [f32((8, 128))],
            compiler_params=_cparams(),
        )(*o_prev, *a_args, w, sel8,
          pools[l - 1]['Wp']['bold'], pools[l - 1]['Wp']['dti'],
          cn['W']['bold'], cn['W']['dti'],
          *att[l][0], *att[l][1], kWs[l], kbs[l], qs[l])

    r1 = stage(1, N1, P1, w1, o1[0:4], o1[4:8], [_perg(bb, (N1, N1))] * 4)
    Pb1, Pd1, x1 = r1[0], r1[1], r1[2]
    An1, o2 = r1[3:7], r1[7:11]
    w2 = _w_pack(r1[11], nb, P1)

    r2 = stage(2, P1, P2, w2, o2, An1, [_perg(bb, (P1, P1))] * 4)
    Pb2, Pd2, x2 = r2[0], r2[1], r2[2]
    o3 = r2[7:11]
    w3 = _w_pack(r2[11], nb, P2)

    r3 = pl.pallas_call(
        functools.partial(_k3b_body, bb),
        grid=(nb // bb,),
        in_specs=[_perg(bb, (P2, HID))] * 4 + [_full((8, 128))]
                 + [_full((HID, P3))] * 2,
        out_specs=[_perg(bb, (P2, P3))] * 2 + [_perg(bb, (1, 4 * HID))],
        out_shape=[f32((nb, P2, P3))] * 2 + [f32((nb, 1, 4 * HID))],
        compiler_params=_cparams(),
    )(*o3, w3, pools[2]['Wp']['bold'], pools[2]['Wp']['dti'])
    Pb3, Pd3, x3 = r3

    out, h = pl.pallas_call(
        _k4_body,
        grid=(1,),
        in_specs=[_full((nb, 1, 4 * HID))] * 3
                 + [_full((4 * HID, HID)), _full((1, HID)),
                    _full((HID, HID // 2)), _full((1, HID // 2)),
                    _full((HID // 2, OUT_C)), _full((1, OUT_C))],
        out_specs=[_full((nb, OUT_C)), _full((nb, HID // 2))],
        out_shape=[f32((nb, OUT_C)), f32((nb, HID // 2))],
        compiler_params=_cparams(),
    )(x1, x2, x3,
      params['lin1_W'], params['lin1_b'].reshape(1, HID),
      params['lin2_W'], params['lin2_b'].reshape(1, HID // 2),
      params['lin3_W'], params['lin3_b'].reshape(1, OUT_C))

    return (out, h, Pb1, Pd1, Pb2, Pd2, Pb3, Pd3)


# speedup vs baseline: 1.0182x; 1.0182x over previous
import functools

import jax
import jax.numpy as jnp
from jax.experimental import pallas as pl
from jax.experimental.pallas import tpu as pltpu

B_, N1, IN_C, HID, HEADS, OUT_C, DEG = 256, 90, 90, 128, 8, 2, 16
D = HID // HEADS
P1, P2, P3 = 72, 57, 45
F32 = jnp.float32
ETS = [('bb', 'b', 'b'), ('dd', 'd', 'd'), ('bd', 'b', 'd'), ('db', 'd', 'b')]
SCORE_ETS = ('bb', 'db', 'dd', 'bd')


def _dot(a, b):
    return jnp.dot(a, b, preferred_element_type=F32)


def _dotf(a, b):
    return jnp.dot(a, b, preferred_element_type=F32,
                   precision=jax.lax.Precision.DEFAULT)


def _dgT(a, b):
    return jax.lax.dot_general(a, b, (((0,), (0,)), ((), ())),
                               preferred_element_type=F32)


def _softmax_rows(x):
    m = jnp.max(x, axis=-1, keepdims=True)
    e = jnp.exp(x - m)
    return e / jnp.sum(e, axis=-1, keepdims=True)


def _build_A(srcT, n):
    nw = (n + 31) // 32
    val = jnp.left_shift(jnp.int32(1), jnp.bitwise_and(srcT, 31))
    word = jnp.right_shift(srcT, 5)
    rows = []
    for w in range(nw):
        m = jnp.where(word == w, val, 0)
        r = m[0:8] | m[8:16]
        r = r[0:4] | r[4:8]
        r = r[0:2] | r[2:4]
        rows.append(r[0:1] | r[1:2])
    W = jnp.concatenate(rows, axis=0).T
    bits = jax.lax.broadcasted_iota(jnp.int32, (n, 32), 1)
    segs = [jnp.bitwise_and(jnp.right_shift(W[:, w:w + 1], bits), 1)
            for w in range(nw)]
    return jnp.concatenate(segs, axis=1)[:, :n].astype(F32)


def _attention_et(hs, hd, A, sel8, asrcT, adstT):
    n = hs.shape[0]
    a_s = _dot(hs, asrcT)
    a_d = _dot(hd, adstT)
    c = a_d + jnp.max(a_s, axis=0, keepdims=True)
    c = jnp.maximum(c, 0.2 * c)
    a_sP = jnp.concatenate([a_s.T, jnp.zeros((HEADS, 128 - n), F32)], axis=1)
    y = sel8 * jnp.concatenate([a_sP] * HEADS, axis=1)
    lhs = jnp.concatenate([a_d, jnp.ones((n, HEADS), F32), c], axis=1)
    u2 = _dotf(lhs, jnp.concatenate([sel8, y, -sel8], axis=0))
    u3 = _dotf(lhs, jnp.concatenate([0.2 * sel8, 0.2 * y, -sel8], axis=0))
    a128 = jnp.concatenate([A + 1e-9, jnp.zeros((n, 128 - n), F32)], axis=1)
    e_all = jnp.exp(jnp.maximum(u2, u3)) \
        * jnp.concatenate([a128] * HEADS, axis=1)
    ones = jnp.ones((n, 1), F32)
    zpad = jnp.zeros((128 - n, D + 1), F32)
    cols = []
    for hh in range(HEADS):
        e_h = e_all[:, 128 * hh:128 * (hh + 1)]
        hp = jnp.concatenate(
            [jnp.concatenate([hs[:, hh * D:(hh + 1) * D], ones], axis=1),
             zpad], axis=0)
        oz = _dotf(e_h, hp)
        cols.append(oz[:, 0:D] / oz[:, D:D + 1])
    return jnp.maximum(jnp.concatenate(cols, axis=1), 0.0)


def _conv_a(bb, xb3, xd3, Amap, sel8, Wb, Wd, asrcT, adstT, kW, kb, q,
            o_refs, sc_ref, n):
    h2 = {'b': _dot(xb3.reshape(bb * n, -1), Wb),
          'd': _dot(xd3.reshape(bb * n, -1), Wd)}
    h = {nt: [h2[nt][g * n:(g + 1) * n] for g in range(bb)] for nt in h2}
    o = {}
    for i, (et, s, d) in enumerate(ETS):
        o[et] = [_attention_et(h[s][g], h[d][g], Amap[et][g], sel8,
                               asrcT[et], adstT[et]) for g in range(bb)]
        for g in range(bb):
            o_refs[i][g] = o[et][g]
    big = jnp.concatenate([o[et][g] for et in SCORE_ETS for g in range(bb)],
                          axis=0)
    t = jnp.tanh(_dot(big, kW) + kb) * q
    sums = jnp.sum(t.reshape(4, bb * n, HID), axis=1)
    contrib = jnp.concatenate([sums, jnp.zeros((4, HID), F32)], axis=0)
    b = pl.program_id(0)
    prev = jnp.where(b == 0, jnp.zeros((8, 128), F32), sc_ref[...])
    sc_ref[...] = prev + contrib
    return o


def _combine3(w_ref, o_bb, o_dd, o_bd, o_db):
    xb = jnp.maximum(w_ref[0:1, 0:1] * o_bb[...] + w_ref[1:2, 0:1] * o_db[...], 0.0)
    xd = jnp.maximum(w_ref[2:3, 0:1] * o_dd[...] + w_ref[3:4, 0:1] * o_bd[...], 0.0)
    return xb, xd


def _readout(xnb, xnd):
    parts = [jnp.max(xnb, axis=0, keepdims=True),
             jnp.mean(xnb, axis=0, keepdims=True),
             jnp.max(xnd, axis=0, keepdims=True),
             jnp.mean(xnd, axis=0, keepdims=True)]
    return jnp.concatenate(parts, axis=1)


def _pool(bb, n, p_dim, xb3, xd3, Wp_b, Wp_d, A, Pb_o, Pd_o, xr_o):
    Sb2 = _softmax_rows(_dot(xb3.reshape(bb * n, HID), Wp_b))
    Sd2 = _softmax_rows(_dot(xd3.reshape(bb * n, HID), Wp_d))
    Pb_o[...] = Sb2.reshape(bb, n, p_dim)
    Pd_o[...] = Sd2.reshape(bb, n, p_dim)
    S = {'b': [Sb2[g * n:(g + 1) * n] for g in range(bb)],
         'd': [Sd2[g * n:(g + 1) * n] for g in range(bb)]}
    xnb, xnd, An = [], [], {et: [] for et, _, _ in ETS}
    for g in range(bb):
        St = {'b': S['b'][g].T, 'd': S['d'][g].T}
        xnb.append(_dot(St['b'], xb3[g]))
        xnd.append(_dot(St['d'], xd3[g]))
        xr_o[g] = _readout(xnb[-1], xnd[-1])
        for et, s, d in ETS:
            An[et].append(_dot(St[d], _dot(A[et][g], S[s][g])))
    return xnb, xnd, An


def _k1a_body(bb,
              xb_ref, xd_ref, s_bb, s_dd, s_bd, s_db, sel8,
              Wb, Wd, as_bb, as_dd, as_bd, as_db, ad_bb, ad_dd, ad_bd, ad_db,
              kW, kb, q,
              o_bb, o_dd, o_bd, o_db, sc_ref):
    b = pl.program_id(0)
    src = {'bb': s_bb, 'dd': s_dd, 'bd': s_bd, 'db': s_db}
    Amap = {et: [_build_A(src[et][g] - (b * bb + g) * N1, N1)
                 for g in range(bb)] for et, _, _ in ETS}
    asrcT = {'bb': as_bb[...], 'dd': as_dd[...], 'bd': as_bd[...], 'db': as_db[...]}
    adstT = {'bb': ad_bb[...], 'dd': ad_dd[...], 'bd': ad_bd[...], 'db': ad_db[...]}
    _conv_a(bb, xb_ref[...], xd_ref[...], Amap, sel8[...], Wb[...], Wd[...],
            asrcT, adstT,
            kW[...], kb[...], q[...], [o_bb, o_dd, o_bd, o_db], sc_ref, N1)


def _stage_body(bb, n, p_dim, rebuild_A,
                o1_bb, o1_dd, o1_bd, o1_db, a_bb, a_dd, a_bd, a_db, w_ref,
                sel8, Wp_b, Wp_d, Wb, Wd,
                as_bb, as_dd, as_bd, as_db, ad_bb, ad_dd, ad_bd, ad_db,
                kW, kb, q,
                Pb_o, Pd_o, xr_o, An_bb, An_dd, An_bd, An_db,
                o2_bb, o2_dd, o2_bd, o2_db, sc_ref):
    b = pl.program_id(0)
    a_in = {'bb': a_bb, 'dd': a_dd, 'bd': a_bd, 'db': a_db}
    if rebuild_A:
        A = {et: [_build_A(a_in[et][g] - (b * bb + g) * n, n)
                  for g in range(bb)] for et, _, _ in ETS}
    else:
        A = {et: [a_in[et][g] for g in range(bb)] for et, _, _ in ETS}
    xb3, xd3 = _combine3(w_ref, o1_bb, o1_dd, o1_bd, o1_db)
    xnb, xnd, An = _pool(bb, n, p_dim, xb3, xd3, Wp_b[...], Wp_d[...], A,
                         Pb_o, Pd_o, xr_o)
    An_o = {'bb': An_bb, 'dd': An_dd, 'bd': An_bd, 'db': An_db}
    for et, _, _ in ETS:
        for g in range(bb):
            An_o[et][g] = An[et][g]
    asrcT = {'bb': as_bb[...], 'dd': as_dd[...], 'bd': as_bd[...], 'db': as_db[...]}
    adstT = {'bb': ad_bb[...], 'dd': ad_dd[...], 'bd': ad_bd[...], 'db': ad_db[...]}
    xnb3 = jnp.concatenate([x[None] for x in xnb], axis=0)
    xnd3 = jnp.concatenate([x[None] for x in xnd], axis=0)
    _conv_a(bb, xnb3, xnd3, An, sel8[...], Wb[...], Wd[...], asrcT, adstT,
            kW[...], kb[...], q[...], [o2_bb, o2_dd, o2_bd, o2_db], sc_ref,
            p_dim)


def _k3b_body(bb, o_bb, o_dd, o_bd, o_db, w_ref, Wp_b, Wp_d, Pb_o, Pd_o, xr_o):
    n, p_dim = P2, P3
    xb3, xd3 = _combine3(w_ref, o_bb, o_dd, o_bd, o_db)
    Sb2 = _softmax_rows(_dot(xb3.reshape(bb * n, HID), Wp_b[...]))
    Sd2 = _softmax_rows(_dot(xd3.reshape(bb * n, HID), Wp_d[...]))
    Pb_o[...] = Sb2.reshape(bb, n, p_dim)
    Pd_o[...] = Sd2.reshape(bb, n, p_dim)
    for g in range(bb):
        xnb = _dot(Sb2[g * n:(g + 1) * n].T, xb3[g])
        xnd = _dot(Sd2[g * n:(g + 1) * n].T, xd3[g])
        xr_o[g] = _readout(xnb, xnd)


def _k4_body(x1_ref, x2_ref, x3_ref, l1W, l1b, l2W, l2b, l3W, l3b,
             out_o, h_o):
    nb = x1_ref.shape[0]
    s = (x1_ref[...] + x2_ref[...] + x3_ref[...]).reshape(nb, 4 * HID)
    s = s - jnp.mean(s, axis=0, keepdims=True)
    rn = jnp.sqrt(1e-6 + jnp.sum(s * s, axis=1, keepdims=True))
    feat = 100.0 * s / rn
    h1 = jnp.maximum(_dot(feat, l1W[...]) + l1b[...], 0.0)
    h2 = jnp.maximum(_dot(h1, l2W[...]) + l2b[...], 0.0)
    out_o[...] = _dot(h2, l3W[...]) + l3b[...]
    h_o[...] = h2


def _att_mat(a):
    m = jnp.zeros((HID, HEADS), F32)
    return m.at[jnp.arange(HID), jnp.arange(HID) // D].set(a.reshape(HID))


def _w_pack(score_out, nb, n):
    s = jnp.sum(score_out, axis=1)[0:4] / (nb * n)
    w = jnp.concatenate([jax.nn.softmax(s[0:2]), jax.nn.softmax(s[2:4])])
    w = jnp.concatenate([w, jnp.zeros((4,), F32)])
    return jnp.broadcast_to(w[:, None], (8, 128))


def _full(shape):
    nd = len(shape)
    return pl.BlockSpec(shape, lambda b, _nd=nd: (0,) * _nd)


def _perg(bb, shape):
    nd = len(shape)
    return pl.BlockSpec((bb,) + shape, lambda b, _nd=nd: (b,) + (0,) * _nd)


def _cparams():
    return pltpu.CompilerParams(dimension_semantics=("arbitrary",))


def kernel(x_bold, x_dti, ei_bb, ei_dd, ei_bd, ei_db, params):
    nb = x_bold.shape[0] // N1
    bb = 4 if nb % 4 == 0 else 1
    f32 = lambda shape: jax.ShapeDtypeStruct(shape, F32)

    srcs = [ei[0].astype(jnp.int32).reshape(nb, N1, DEG).transpose(0, 2, 1)
            for ei in (ei_bb, ei_dd, ei_bd, ei_db)]
    xb = x_bold.reshape(nb, N1, IN_C)
    xd = x_dti.reshape(nb, N1, IN_C)

    convs = [params['conv%d' % (l + 1)] for l in range(3)]
    pools = [params['pool%d' % (l + 1)] for l in range(3)]
    att = []
    for c in convs:
        att.append(([_att_mat(c['att_src'][et]) for et, _, _ in ETS],
                    [_att_mat(c['att_dst'][et]) for et, _, _ in ETS]))
    kWs = [c['k_W'] for c in convs]
    kbs = [c['k_b'].reshape(1, HID) for c in convs]
    qs = [c['q'].reshape(1, HID) for c in convs]
    r8 = jnp.arange(HEADS)[:, None]
    sel8 = (jnp.arange(HEADS * 128)[None, :] // 128 == r8).astype(F32)

    c = convs[0]
    o1 = pl.pallas_call(
        functools.partial(_k1a_body, bb),
        grid=(nb // bb,),
        in_specs=[_perg(bb, (N1, IN_C))] * 2 + [_perg(bb, (DEG, N1))] * 4
                 + [_full((HEADS, 1024))]
                 + [_full((IN_C, HID))] * 2 + [_full((HID, HEADS))] * 8
                 + [_full((HID, HID)), _full((1, HID)), _full((1, HID))],
        out_specs=[_perg(bb, (N1, HID))] * 4 + [_full((8, 128))],
        out_shape=[f32((nb, N1, HID))] * 4 + [f32((8, 128))],
        compiler_params=_cparams(),
    )(xb, xd, *srcs, sel8, c['W']['bold'], c['W']['dti'],
      *att[0][0], *att[0][1], kWs[0], kbs[0], qs[0])
    w1 = _w_pack(o1[4], nb, N1)

    def stage(l, n, p_dim, w, o_prev, a_args, a_specs, rebuild):
        cn = convs[l]
        body = functools.partial(_stage_body, bb, n, p_dim, rebuild)
        return pl.pallas_call(
            body,
            grid=(nb // bb,),
            in_specs=[_perg(bb, (n, HID))] * 4 + a_specs + [_full((8, 128))]
                     + [_full((HEADS, 1024))]
                     + [_full((HID, p_dim))] * 2 + [_full((HID, HID))] * 2
                     + [_full((HID, HEADS))] * 8
                     + [_full((HID, HID)), _full((1, HID)), _full((1, HID))],
            out_specs=[_perg(bb, (n, p_dim))] * 2 + [_perg(bb, (1, 4 * HID))]
                      + [_perg(bb, (p_dim, p_dim))] * 4
                      + [_perg(bb, (p_dim, HID))] * 4 + [_full((8, 128))],
            out_shape=[f32((nb, n, p_dim))] * 2 + [f32((nb, 1, 4 * HID))]
                      + [f32((nb, p_dim, p_dim))] * 4
                      + [f32((nb, p_dim, HID))] * 4 + [f32((8, 128))],
            compiler_params=_cparams(),
        )(*o_prev, *a_args, w, sel8,
          pools[l - 1]['Wp']['bold'], pools[l - 1]['Wp']['dti'],
          cn['W']['bold'], cn['W']['dti'],
          *att[l][0], *att[l][1], kWs[l], kbs[l], qs[l])

    r1 = stage(1, N1, P1, w1, o1[0:4], srcs, [_perg(bb, (DEG, N1))] * 4, True)
    Pb1, Pd1, x1 = r1[0], r1[1], r1[2]
    An1, o2 = r1[3:7], r1[7:11]
    w2 = _w_pack(r1[11], nb, P1)

    r2 = stage(2, P1, P2, w2, o2, An1, [_perg(bb, (P1, P1))] * 4, False)
    Pb2, Pd2, x2 = r2[0], r2[1], r2[2]
    o3 = r2[7:11]
    w3 = _w_pack(r2[11], nb, P2)

    r3 = pl.pallas_call(
        functools.partial(_k3b_body, bb),
        grid=(nb // bb,),
        in_specs=[_perg(bb, (P2, HID))] * 4 + [_full((8, 128))]
                 + [_full((HID, P3))] * 2,
        out_specs=[_perg(bb, (P2, P3))] * 2 + [_perg(bb, (1, 4 * HID))],
        out_shape=[f32((nb, P2, P3))] * 2 + [f32((nb, 1, 4 * HID))],
        compiler_params=_cparams(),
    )(*o3, w3, pools[2]['Wp']['bold'], pools[2]['Wp']['dti'])
    Pb3, Pd3, x3 = r3

    out, h = pl.pallas_call(
        _k4_body,
        grid=(1,),
        in_specs=[_full((nb, 1, 4 * HID))] * 3
                 + [_full((4 * HID, HID)), _full((1, HID)),
                    _full((HID, HID // 2)), _full((1, HID // 2)),
                    _full((HID // 2, OUT_C)), _full((1, OUT_C))],
        out_specs=[_full((nb, OUT_C)), _full((nb, HID // 2))],
        out_shape=[f32((nb, OUT_C)), f32((nb, HID // 2))],
        compiler_params=_cparams(),
    )(x1, x2, x3,
      params['lin1_W'], params['lin1_b'].reshape(1, HID),
      params['lin2_W'], params['lin2_b'].reshape(1, HID // 2),
      params['lin3_W'], params['lin3_b'].reshape(1, OUT_C))

    return (out, h, Pb1, Pd1, Pb2, Pd2, Pb3, Pd3)


# speedup vs baseline: 1.1080x; 1.0882x over previous
import functools

import jax
import jax.numpy as jnp
from jax.experimental import pallas as pl
from jax.experimental.pallas import tpu as pltpu

B_, N1, IN_C, HID, HEADS, OUT_C, DEG = 256, 90, 90, 128, 8, 2, 16
D = HID // HEADS
P1, P2, P3 = 72, 57, 45
F32 = jnp.float32
ETS = [('bb', 'b', 'b'), ('dd', 'd', 'd'), ('bd', 'b', 'd'), ('db', 'd', 'b')]
SCORE_ETS = ('bb', 'db', 'dd', 'bd')


def _dot(a, b):
    return jnp.dot(a, b, preferred_element_type=F32)


def _dgT(a, b):
    return jax.lax.dot_general(a, b, (((0,), (0,)), ((), ())),
                               preferred_element_type=F32)


def _softmax_rows(x):
    m = jnp.max(x, axis=-1, keepdims=True)
    e = jnp.exp(x - m)
    return e / jnp.sum(e, axis=-1, keepdims=True)


def _build_A(srcT, n):
    nw = (n + 31) // 32
    val = jnp.left_shift(jnp.int32(1), jnp.bitwise_and(srcT, 31))
    word = jnp.right_shift(srcT, 5)
    rows = []
    for w in range(nw):
        m = jnp.where(word == w, val, 0)
        r = m[0:8] | m[8:16]
        r = r[0:4] | r[4:8]
        r = r[0:2] | r[2:4]
        rows.append(r[0:1] | r[1:2])
    W = jnp.concatenate(rows, axis=0).T
    bits = jax.lax.broadcasted_iota(jnp.int32, (n, 32), 1)
    segs = [jnp.bitwise_and(jnp.right_shift(W[:, w:w + 1], bits), 1)
            for w in range(nw)]
    return jnp.concatenate(segs, axis=1)[:, :n].astype(F32)


def _attention_et(hs, hd, A, sel8, asrcT, adstT):
    n = hs.shape[0]
    a_s = _dot(hs, asrcT)
    a_d = _dot(hd, adstT)
    c = a_d + jnp.max(a_s, axis=0, keepdims=True)
    c = jnp.maximum(c, 0.2 * c)
    a_sP = jnp.concatenate([a_s.T, jnp.zeros((HEADS, 128 - n), F32)], axis=1)
    y = sel8 * jnp.concatenate([a_sP] * HEADS, axis=1)
    lhs = jnp.concatenate([a_d, jnp.ones((n, HEADS), F32)], axis=1)
    u = _dot(lhs, jnp.concatenate([sel8, y], axis=0))
    cb = _dot(c, sel8)
    a128 = jnp.concatenate([A + 1e-9, jnp.zeros((n, 128 - n), F32)], axis=1)
    e_all = jnp.exp(jnp.maximum(u, 0.2 * u) - cb) \
        * jnp.concatenate([a128] * HEADS, axis=1)
    ones = jnp.ones((n, 1), F32)
    zpad = jnp.zeros((128 - n, D + 1), F32)
    cols = []
    for hh in range(HEADS):
        e_h = e_all[:, 128 * hh:128 * (hh + 1)]
        hp = jnp.concatenate(
            [jnp.concatenate([hs[:, hh * D:(hh + 1) * D], ones], axis=1),
             zpad], axis=0)
        oz = _dot(e_h, hp)
        cols.append(oz[:, 0:D] / oz[:, D:D + 1])
    return jnp.maximum(jnp.concatenate(cols, axis=1), 0.0)


def _conv_a(bb, xb3, xd3, Amap, sel8, Wb, Wd, asrcT, adstT, kW, kb, q,
            o_refs, sc_ref, n):
    h2 = {'b': _dot(xb3.reshape(bb * n, -1), Wb),
          'd': _dot(xd3.reshape(bb * n, -1), Wd)}
    h = {nt: [h2[nt][g * n:(g + 1) * n] for g in range(bb)] for nt in h2}
    o = {}
    for i, (et, s, d) in enumerate(ETS):
        o[et] = [_attention_et(h[s][g], h[d][g], Amap[et][g], sel8,
                               asrcT[et], adstT[et]) for g in range(bb)]
        for g in range(bb):
            o_refs[i][g] = o[et][g]
    big = jnp.concatenate([o[et][g] for et in SCORE_ETS for g in range(bb)],
                          axis=0)
    t = jnp.tanh(_dot(big, kW) + kb) * q
    sums = jnp.sum(t.reshape(4, bb * n, HID), axis=1)
    contrib = jnp.concatenate([sums, jnp.zeros((4, HID), F32)], axis=0)
    b = pl.program_id(0)
    prev = jnp.where(b == 0, jnp.zeros((8, 128), F32), sc_ref[...])
    sc_ref[...] = prev + contrib
    return o


def _combine3(w_ref, o_bb, o_dd, o_bd, o_db):
    xb = jnp.maximum(w_ref[0:1, 0:1] * o_bb[...] + w_ref[1:2, 0:1] * o_db[...], 0.0)
    xd = jnp.maximum(w_ref[2:3, 0:1] * o_dd[...] + w_ref[3:4, 0:1] * o_bd[...], 0.0)
    return xb, xd


def _readout(xnb, xnd):
    parts = [jnp.max(xnb, axis=0, keepdims=True),
             jnp.mean(xnb, axis=0, keepdims=True),
             jnp.max(xnd, axis=0, keepdims=True),
             jnp.mean(xnd, axis=0, keepdims=True)]
    return jnp.concatenate(parts, axis=1)


def _pool(bb, n, p_dim, xb3, xd3, Wp_b, Wp_d, A, Pb_o, Pd_o, xr_o):
    Sb2 = _softmax_rows(_dot(xb3.reshape(bb * n, HID), Wp_b))
    Sd2 = _softmax_rows(_dot(xd3.reshape(bb * n, HID), Wp_d))
    Pb_o[...] = Sb2.reshape(bb, n, p_dim)
    Pd_o[...] = Sd2.reshape(bb, n, p_dim)
    S = {'b': [Sb2[g * n:(g + 1) * n] for g in range(bb)],
         'd': [Sd2[g * n:(g + 1) * n] for g in range(bb)]}
    xnb, xnd, An = [], [], {et: [] for et, _, _ in ETS}
    for g in range(bb):
        St = {'b': S['b'][g].T, 'd': S['d'][g].T}
        xnb.append(_dot(St['b'], xb3[g]))
        xnd.append(_dot(St['d'], xd3[g]))
        xr_o[g] = _readout(xnb[-1], xnd[-1])
        for et, s, d in ETS:
            An[et].append(_dot(St[d], _dot(A[et][g], S[s][g])))
    return xnb, xnd, An


def _k1a_body(bb,
              xb_ref, xd_ref, s_bb, s_dd, s_bd, s_db, sel8,
              Wb, Wd, as_bb, as_dd, as_bd, as_db, ad_bb, ad_dd, ad_bd, ad_db,
              kW, kb, q,
              o_bb, o_dd, o_bd, o_db, sc_ref):
    b = pl.program_id(0)
    src = {'bb': s_bb, 'dd': s_dd, 'bd': s_bd, 'db': s_db}
    Amap = {et: [_build_A(src[et][g] - (b * bb + g) * N1, N1)
                 for g in range(bb)] for et, _, _ in ETS}
    asrcT = {'bb': as_bb[...], 'dd': as_dd[...], 'bd': as_bd[...], 'db': as_db[...]}
    adstT = {'bb': ad_bb[...], 'dd': ad_dd[...], 'bd': ad_bd[...], 'db': ad_db[...]}
    _conv_a(bb, xb_ref[...], xd_ref[...], Amap, sel8[...], Wb[...], Wd[...],
            asrcT, adstT,
            kW[...], kb[...], q[...], [o_bb, o_dd, o_bd, o_db], sc_ref, N1)


def _stage_body(bb, n, p_dim, rebuild_A,
                o1_bb, o1_dd, o1_bd, o1_db, a_bb, a_dd, a_bd, a_db, w_ref,
                sel8, Wp_b, Wp_d, Wb, Wd,
                as_bb, as_dd, as_bd, as_db, ad_bb, ad_dd, ad_bd, ad_db,
                kW, kb, q,
                Pb_o, Pd_o, xr_o, An_bb, An_dd, An_bd, An_db,
                o2_bb, o2_dd, o2_bd, o2_db, sc_ref):
    b = pl.program_id(0)
    a_in = {'bb': a_bb, 'dd': a_dd, 'bd': a_bd, 'db': a_db}
    if rebuild_A:
        A = {et: [_build_A(a_in[et][g] - (b * bb + g) * n, n)
                  for g in range(bb)] for et, _, _ in ETS}
    else:
        A = {et: [a_in[et][g] for g in range(bb)] for et, _, _ in ETS}
    xb3, xd3 = _combine3(w_ref, o1_bb, o1_dd, o1_bd, o1_db)
    xnb, xnd, An = _pool(bb, n, p_dim, xb3, xd3, Wp_b[...], Wp_d[...], A,
                         Pb_o, Pd_o, xr_o)
    An_o = {'bb': An_bb, 'dd': An_dd, 'bd': An_bd, 'db': An_db}
    for et, _, _ in ETS:
        for g in range(bb):
            An_o[et][g] = An[et][g]
    asrcT = {'bb': as_bb[...], 'dd': as_dd[...], 'bd': as_bd[...], 'db': as_db[...]}
    adstT = {'bb': ad_bb[...], 'dd': ad_dd[...], 'bd': ad_bd[...], 'db': ad_db[...]}
    xnb3 = jnp.concatenate([x[None] for x in xnb], axis=0)
    xnd3 = jnp.concatenate([x[None] for x in xnd], axis=0)
    _conv_a(bb, xnb3, xnd3, An, sel8[...], Wb[...], Wd[...], asrcT, adstT,
            kW[...], kb[...], q[...], [o2_bb, o2_dd, o2_bd, o2_db], sc_ref,
            p_dim)


def _k3b_body(bb, o_bb, o_dd, o_bd, o_db, w_ref, Wp_b, Wp_d, Pb_o, Pd_o, xr_o):
    n, p_dim = P2, P3
    xb3, xd3 = _combine3(w_ref, o_bb, o_dd, o_bd, o_db)
    Sb2 = _softmax_rows(_dot(xb3.reshape(bb * n, HID), Wp_b[...]))
    Sd2 = _softmax_rows(_dot(xd3.reshape(bb * n, HID), Wp_d[...]))
    Pb_o[...] = Sb2.reshape(bb, n, p_dim)
    Pd_o[...] = Sd2.reshape(bb, n, p_dim)
    for g in range(bb):
        xnb = _dot(Sb2[g * n:(g + 1) * n].T, xb3[g])
        xnd = _dot(Sd2[g * n:(g + 1) * n].T, xd3[g])
        xr_o[g] = _readout(xnb, xnd)


def _k4_body(x1_ref, x2_ref, x3_ref, l1W, l1b, l2W, l2b, l3W, l3b,
             out_o, h_o):
    nb = x1_ref.shape[0]
    s = (x1_ref[...] + x2_ref[...] + x3_ref[...]).reshape(nb, 4 * HID)
    s = s - jnp.mean(s, axis=0, keepdims=True)
    rn = jnp.sqrt(1e-6 + jnp.sum(s * s, axis=1, keepdims=True))
    feat = 100.0 * s / rn
    h1 = jnp.maximum(_dot(feat, l1W[...]) + l1b[...], 0.0)
    h2 = jnp.maximum(_dot(h1, l2W[...]) + l2b[...], 0.0)
    out_o[...] = _dot(h2, l3W[...]) + l3b[...]
    h_o[...] = h2


def _att_mat(a):
    m = jnp.zeros((HID, HEADS), F32)
    return m.at[jnp.arange(HID), jnp.arange(HID) // D].set(a.reshape(HID))


def _w_pack(score_out, nb, n):
    s = jnp.sum(score_out, axis=1)[0:4] / (nb * n)
    w = jnp.concatenate([jax.nn.softmax(s[0:2]), jax.nn.softmax(s[2:4])])
    w = jnp.concatenate([w, jnp.zeros((4,), F32)])
    return jnp.broadcast_to(w[:, None], (8, 128))


def _full(shape):
    nd = len(shape)
    return pl.BlockSpec(shape, lambda b, _nd=nd: (0,) * _nd)


def _perg(bb, shape):
    nd = len(shape)
    return pl.BlockSpec((bb,) + shape, lambda b, _nd=nd: (b,) + (0,) * _nd)


def _cparams():
    return pltpu.CompilerParams(dimension_semantics=("arbitrary",))


def kernel(x_bold, x_dti, ei_bb, ei_dd, ei_bd, ei_db, params):
    nb = x_bold.shape[0] // N1
    bb = 4 if nb % 4 == 0 else 1
    bs = 8 if nb % 8 == 0 else bb
    f32 = lambda shape: jax.ShapeDtypeStruct(shape, F32)

    srcs = [ei[0].astype(jnp.int32).reshape(nb, N1, DEG).transpose(0, 2, 1)
            for ei in (ei_bb, ei_dd, ei_bd, ei_db)]
    xb = x_bold.reshape(nb, N1, IN_C)
    xd = x_dti.reshape(nb, N1, IN_C)

    convs = [params['conv%d' % (l + 1)] for l in range(3)]
    pools = [params['pool%d' % (l + 1)] for l in range(3)]
    att = []
    for c in convs:
        att.append(([_att_mat(c['att_src'][et]) for et, _, _ in ETS],
                    [_att_mat(c['att_dst'][et]) for et, _, _ in ETS]))
    kWs = [c['k_W'] for c in convs]
    kbs = [c['k_b'].reshape(1, HID) for c in convs]
    qs = [c['q'].reshape(1, HID) for c in convs]
    r8 = jnp.arange(HEADS)[:, None]
    sel8 = (jnp.arange(HEADS * 128)[None, :] // 128 == r8).astype(F32)

    c = convs[0]
    o1 = pl.pallas_call(
        functools.partial(_k1a_body, bb),
        grid=(nb // bb,),
        in_specs=[_perg(bb, (N1, IN_C))] * 2 + [_perg(bb, (DEG, N1))] * 4
                 + [_full((HEADS, 1024))]
                 + [_full((IN_C, HID))] * 2 + [_full((HID, HEADS))] * 8
                 + [_full((HID, HID)), _full((1, HID)), _full((1, HID))],
        out_specs=[_perg(bb, (N1, HID))] * 4 + [_full((8, 128))],
        out_shape=[f32((nb, N1, HID))] * 4 + [f32((8, 128))],
        compiler_params=_cparams(),
    )(xb, xd, *srcs, sel8, c['W']['bold'], c['W']['dti'],
      *att[0][0], *att[0][1], kWs[0], kbs[0], qs[0])
    w1 = _w_pack(o1[4], nb, N1)

    def stage(l, n, p_dim, w, o_prev, a_args, a_specs, rebuild):
        cn = convs[l]
        body = functools.partial(_stage_body, bs, n, p_dim, rebuild)
        return pl.pallas_call(
            body,
            grid=(nb // bs,),
            in_specs=[_perg(bs, (n, HID))] * 4 + a_specs + [_full((8, 128))]
                     + [_full((HEADS, 1024))]
                     + [_full((HID, p_dim))] * 2 + [_full((HID, HID))] * 2
                     + [_full((HID, HEADS))] * 8
                     + [_full((HID, HID)), _full((1, HID)), _full((1, HID))],
            out_specs=[_perg(bs, (n, p_dim))] * 2 + [_perg(bs, (1, 4 * HID))]
                      + [_perg(bs, (p_dim, p_dim))] * 4
                      + [_perg(bs, (p_dim, HID))] * 4 + [_full((8, 128))],
            out_shape=[f32((nb, n, p_dim))] * 2 + [f32((nb, 1, 4 * HID))]
                      + [f32((nb, p_dim, p_dim))] * 4
                      + [f32((nb, p_dim, HID))] * 4 + [f32((8, 128))],
            compiler_params=_cparams(),
        )(*o_prev, *a_args, w, sel8,
          pools[l - 1]['Wp']['bold'], pools[l - 1]['Wp']['dti'],
          cn['W']['bold'], cn['W']['dti'],
          *att[l][0], *att[l][1], kWs[l], kbs[l], qs[l])

    r1 = stage(1, N1, P1, w1, o1[0:4], srcs, [_perg(bs, (DEG, N1))] * 4, True)
    Pb1, Pd1, x1 = r1[0], r1[1], r1[2]
    An1, o2 = r1[3:7], r1[7:11]
    w2 = _w_pack(r1[11], nb, P1)

    r2 = stage(2, P1, P2, w2, o2, An1, [_perg(bs, (P1, P1))] * 4, False)
    Pb2, Pd2, x2 = r2[0], r2[1], r2[2]
    o3 = r2[7:11]
    w3 = _w_pack(r2[11], nb, P2)

    r3 = pl.pallas_call(
        functools.partial(_k3b_body, bs),
        grid=(nb // bs,),
        in_specs=[_perg(bs, (P2, HID))] * 4 + [_full((8, 128))]
                 + [_full((HID, P3))] * 2,
        out_specs=[_perg(bs, (P2, P3))] * 2 + [_perg(bs, (1, 4 * HID))],
        out_shape=[f32((nb, P2, P3))] * 2 + [f32((nb, 1, 4 * HID))],
        compiler_params=_cparams(),
    )(*o3, w3, pools[2]['Wp']['bold'], pools[2]['Wp']['dti'])
    Pb3, Pd3, x3 = r3

    out, h = pl.pallas_call(
        _k4_body,
        grid=(1,),
        in_specs=[_full((nb, 1, 4 * HID))] * 3
                 + [_full((4 * HID, HID)), _full((1, HID)),
                    _full((HID, HID // 2)), _full((1, HID // 2)),
                    _full((HID // 2, OUT_C)), _full((1, OUT_C))],
        out_specs=[_full((nb, OUT_C)), _full((nb, HID // 2))],
        out_shape=[f32((nb, OUT_C)), f32((nb, HID // 2))],
        compiler_params=_cparams(),
    )(x1, x2, x3,
      params['lin1_W'], params['lin1_b'].reshape(1, HID),
      params['lin2_W'], params['lin2_b'].reshape(1, HID // 2),
      params['lin3_W'], params['lin3_b'].reshape(1, OUT_C))

    return (out, h, Pb1, Pd1, Pb2, Pd2, Pb3, Pd3)


# speedup vs baseline: 1.1083x; 1.0003x over previous
import functools

import jax
import jax.numpy as jnp
from jax.experimental import pallas as pl
from jax.experimental.pallas import tpu as pltpu

B_, N1, IN_C, HID, HEADS, OUT_C, DEG = 256, 90, 90, 128, 8, 2, 16
D = HID // HEADS
P1, P2, P3 = 72, 57, 45
F32 = jnp.float32
ETS = [('bb', 'b', 'b'), ('dd', 'd', 'd'), ('bd', 'b', 'd'), ('db', 'd', 'b')]
SCORE_ETS = ('bb', 'db', 'dd', 'bd')


def _dot(a, b):
    return jnp.dot(a, b, preferred_element_type=F32)


def _softmax_rows(x):
    m = jnp.max(x, axis=-1, keepdims=True)
    e = jnp.exp(x - m)
    return e / jnp.sum(e, axis=-1, keepdims=True)


def _build_A(srcT, n):
    nw = (n + 31) // 32
    val = jnp.left_shift(jnp.int32(1), jnp.bitwise_and(srcT, 31))
    word = jnp.right_shift(srcT, 5)
    rows = []
    for w in range(nw):
        m = jnp.where(word == w, val, 0)
        r = m[0:8] | m[8:16]
        r = r[0:4] | r[4:8]
        r = r[0:2] | r[2:4]
        rows.append(r[0:1] | r[1:2])
    W = jnp.concatenate(rows, axis=0).T
    bits = jax.lax.broadcasted_iota(jnp.int32, (n, 32), 1)
    segs = [jnp.bitwise_and(jnp.right_shift(W[:, w:w + 1], bits), 1)
            for w in range(nw)]
    return jnp.concatenate(segs, axis=1)[:, :n].astype(F32)


def _attention_et(hs, hd, A, sel8, asrcT, adstT):
    n = hs.shape[0]
    a_s = _dot(hs, asrcT)
    a_d = _dot(hd, adstT)
    c = a_d + jnp.max(a_s, axis=0, keepdims=True)
    c = jnp.maximum(c, 0.2 * c)
    a_sP = jnp.concatenate([a_s.T, jnp.zeros((HEADS, 128 - n), F32)], axis=1)
    y = sel8 * jnp.concatenate([a_sP] * HEADS, axis=1)
    lhs = jnp.concatenate([a_d, jnp.ones((n, HEADS), F32)], axis=1)
    u = _dot(lhs, jnp.concatenate([sel8, y], axis=0))
    cb = _dot(c, sel8)
    a128 = jnp.concatenate([A + 1e-9, jnp.zeros((n, 128 - n), F32)], axis=1)
    e_all = jnp.exp(jnp.maximum(u, 0.2 * u) - cb) \
        * jnp.concatenate([a128] * HEADS, axis=1)
    ones = jnp.ones((n, 1), F32)
    zpad = jnp.zeros((128 - n, D + 1), F32)
    cols = []
    for hh in range(HEADS):
        e_h = e_all[:, 128 * hh:128 * (hh + 1)]
        hp = jnp.concatenate(
            [jnp.concatenate([hs[:, hh * D:(hh + 1) * D], ones], axis=1),
             zpad], axis=0)
        oz = _dot(e_h, hp)
        cols.append(oz[:, 0:D] / oz[:, D:D + 1])
    return jnp.maximum(jnp.concatenate(cols, axis=1), 0.0)


def _conv_a(bb, xb3, xd3, Amap, sel8, Wb, Wd, asrcT, adstT, kW, kb, q,
            o_refs, sc_ref, n):
    h2 = {'b': _dot(xb3.reshape(bb * n, -1), Wb),
          'd': _dot(xd3.reshape(bb * n, -1), Wd)}
    h = {nt: [h2[nt][g * n:(g + 1) * n] for g in range(bb)] for nt in h2}
    o = {}
    for i, (et, s, d) in enumerate(ETS):
        o[et] = [_attention_et(h[s][g], h[d][g], Amap[et][g], sel8,
                               asrcT[et], adstT[et]) for g in range(bb)]
        for g in range(bb):
            o_refs[i][g] = o[et][g]
    big = jnp.concatenate([o[et][g] for et in SCORE_ETS for g in range(bb)],
                          axis=0)
    t = jnp.tanh(_dot(big, kW) + kb) * q
    sums = jnp.sum(t.reshape(4, bb * n, HID), axis=1)
    contrib = jnp.concatenate([sums, jnp.zeros((4, HID), F32)], axis=0)
    b = pl.program_id(0)
    prev = jnp.where(b == 0, jnp.zeros((8, 128), F32), sc_ref[...])
    sc_ref[...] = prev + contrib
    return o


def _combine3(w_ref, o_bb, o_dd, o_bd, o_db):
    xb = jnp.maximum(w_ref[0:1, 0:1] * o_bb[...] + w_ref[1:2, 0:1] * o_db[...], 0.0)
    xd = jnp.maximum(w_ref[2:3, 0:1] * o_dd[...] + w_ref[3:4, 0:1] * o_bd[...], 0.0)
    return xb, xd


def _readout(xnb, xnd):
    parts = [jnp.max(xnb, axis=0, keepdims=True),
             jnp.mean(xnb, axis=0, keepdims=True),
             jnp.max(xnd, axis=0, keepdims=True),
             jnp.mean(xnd, axis=0, keepdims=True)]
    return jnp.concatenate(parts, axis=1)


def _pool(bb, n, p_dim, xb3, xd3, Wp_b, Wp_d, A, Pb_o, Pd_o, xr_o):
    Sb2 = _softmax_rows(_dot(xb3.reshape(bb * n, HID), Wp_b))
    Sd2 = _softmax_rows(_dot(xd3.reshape(bb * n, HID), Wp_d))
    Pb_o[...] = Sb2.reshape(bb, n, p_dim)
    Pd_o[...] = Sd2.reshape(bb, n, p_dim)
    S = {'b': [Sb2[g * n:(g + 1) * n] for g in range(bb)],
         'd': [Sd2[g * n:(g + 1) * n] for g in range(bb)]}
    xnb, xnd, An = [], [], {et: [] for et, _, _ in ETS}
    for g in range(bb):
        St = {'b': S['b'][g].T, 'd': S['d'][g].T}
        xnb.append(_dot(St['b'], xb3[g]))
        xnd.append(_dot(St['d'], xd3[g]))
        xr_o[g] = _readout(xnb[-1], xnd[-1])
        for et, s, d in ETS:
            An[et].append(_dot(St[d], _dot(A[et][g], S[s][g])))
    return xnb, xnd, An


def _k1a_body(bb,
              xb_ref, xd_ref, s_bb, s_dd, s_bd, s_db, sel8,
              Wb, Wd, as_bb, as_dd, as_bd, as_db, ad_bb, ad_dd, ad_bd, ad_db,
              kW, kb, q,
              o_bb, o_dd, o_bd, o_db, sc_ref):
    b = pl.program_id(0)
    src = {'bb': s_bb, 'dd': s_dd, 'bd': s_bd, 'db': s_db}
    Amap = {et: [_build_A(src[et][g] - (b * bb + g) * N1, N1)
                 for g in range(bb)] for et, _, _ in ETS}
    asrcT = {'bb': as_bb[...], 'dd': as_dd[...], 'bd': as_bd[...], 'db': as_db[...]}
    adstT = {'bb': ad_bb[...], 'dd': ad_dd[...], 'bd': ad_bd[...], 'db': ad_db[...]}
    _conv_a(bb, xb_ref[...], xd_ref[...], Amap, sel8[...], Wb[...], Wd[...],
            asrcT, adstT,
            kW[...], kb[...], q[...], [o_bb, o_dd, o_bd, o_db], sc_ref, N1)


def _stage_body(bb, n, p_dim, rebuild_A,
                o1_bb, o1_dd, o1_bd, o1_db, a_bb, a_dd, a_bd, a_db, w_ref,
                sel8, Wp_b, Wp_d, Wb, Wd,
                as_bb, as_dd, as_bd, as_db, ad_bb, ad_dd, ad_bd, ad_db,
                kW, kb, q,
                Pb_o, Pd_o, xr_o, An_bb, An_dd, An_bd, An_db,
                o2_bb, o2_dd, o2_bd, o2_db, sc_ref):
    b = pl.program_id(0)
    a_in = {'bb': a_bb, 'dd': a_dd, 'bd': a_bd, 'db': a_db}
    if rebuild_A:
        A = {et: [_build_A(a_in[et][g] - (b * bb + g) * n, n)
                  for g in range(bb)] for et, _, _ in ETS}
    else:
        A = {et: [a_in[et][g] for g in range(bb)] for et, _, _ in ETS}
    xb3, xd3 = _combine3(w_ref, o1_bb, o1_dd, o1_bd, o1_db)
    xnb, xnd, An = _pool(bb, n, p_dim, xb3, xd3, Wp_b[...], Wp_d[...], A,
                         Pb_o, Pd_o, xr_o)
    An_o = {'bb': An_bb, 'dd': An_dd, 'bd': An_bd, 'db': An_db}
    for et, _, _ in ETS:
        for g in range(bb):
            An_o[et][g] = An[et][g]
    asrcT = {'bb': as_bb[...], 'dd': as_dd[...], 'bd': as_bd[...], 'db': as_db[...]}
    adstT = {'bb': ad_bb[...], 'dd': ad_dd[...], 'bd': ad_bd[...], 'db': ad_db[...]}
    xnb3 = jnp.concatenate([x[None] for x in xnb], axis=0)
    xnd3 = jnp.concatenate([x[None] for x in xnd], axis=0)
    _conv_a(bb, xnb3, xnd3, An, sel8[...], Wb[...], Wd[...], asrcT, adstT,
            kW[...], kb[...], q[...], [o2_bb, o2_dd, o2_bd, o2_db], sc_ref,
            p_dim)


def _k3b_body(bb, o_bb, o_dd, o_bd, o_db, w_ref, Wp_b, Wp_d, Pb_o, Pd_o, xr_o):
    n, p_dim = P2, P3
    xb3, xd3 = _combine3(w_ref, o_bb, o_dd, o_bd, o_db)
    Sb2 = _softmax_rows(_dot(xb3.reshape(bb * n, HID), Wp_b[...]))
    Sd2 = _softmax_rows(_dot(xd3.reshape(bb * n, HID), Wp_d[...]))
    Pb_o[...] = Sb2.reshape(bb, n, p_dim)
    Pd_o[...] = Sd2.reshape(bb, n, p_dim)
    for g in range(bb):
        xnb = _dot(Sb2[g * n:(g + 1) * n].T, xb3[g])
        xnd = _dot(Sd2[g * n:(g + 1) * n].T, xd3[g])
        xr_o[g] = _readout(xnb, xnd)


def _k4_body(x1_ref, x2_ref, x3_ref, l1W, l1b, l2W, l2b, l3W, l3b,
             out_o, h_o):
    nb = x1_ref.shape[0]
    s = (x1_ref[...] + x2_ref[...] + x3_ref[...]).reshape(nb, 4 * HID)
    s = s - jnp.mean(s, axis=0, keepdims=True)
    rn = jnp.sqrt(1e-6 + jnp.sum(s * s, axis=1, keepdims=True))
    feat = 100.0 * s / rn
    h1 = jnp.maximum(_dot(feat, l1W[...]) + l1b[...], 0.0)
    h2 = jnp.maximum(_dot(h1, l2W[...]) + l2b[...], 0.0)
    out_o[...] = _dot(h2, l3W[...]) + l3b[...]
    h_o[...] = h2


def _att_mat(a):
    m = jnp.zeros((HID, HEADS), F32)
    return m.at[jnp.arange(HID), jnp.arange(HID) // D].set(a.reshape(HID))


def _w_pack(score_out, nb, n):
    s = jnp.sum(score_out, axis=1)[0:4] / (nb * n)
    w = jnp.concatenate([jax.nn.softmax(s[0:2]), jax.nn.softmax(s[2:4])])
    w = jnp.concatenate([w, jnp.zeros((4,), F32)])
    return jnp.broadcast_to(w[:, None], (8, 128))


def _full(shape):
    nd = len(shape)
    return pl.BlockSpec(shape, lambda b, _nd=nd: (0,) * _nd)


def _perg(bb, shape):
    nd = len(shape)
    return pl.BlockSpec((bb,) + shape, lambda b, _nd=nd: (b,) + (0,) * _nd)


def _cparams():
    return pltpu.CompilerParams(dimension_semantics=("arbitrary",))


def kernel(x_bold, x_dti, ei_bb, ei_dd, ei_bd, ei_db, params):
    nb = x_bold.shape[0] // N1
    bb = 4 if nb % 4 == 0 else 1
    bs = 8 if nb % 8 == 0 else bb
    f32 = lambda shape: jax.ShapeDtypeStruct(shape, F32)

    srcs = [ei[0].astype(jnp.int32).reshape(nb, N1, DEG).transpose(0, 2, 1)
            for ei in (ei_bb, ei_dd, ei_bd, ei_db)]
    xb = x_bold.reshape(nb, N1, IN_C)
    xd = x_dti.reshape(nb, N1, IN_C)

    convs = [params['conv%d' % (l + 1)] for l in range(3)]
    pools = [params['pool%d' % (l + 1)] for l in range(3)]
    att = []
    for c in convs:
        att.append(([_att_mat(c['att_src'][et]) for et, _, _ in ETS],
                    [_att_mat(c['att_dst'][et]) for et, _, _ in ETS]))
    kWs = [c['k_W'] for c in convs]
    kbs = [c['k_b'].reshape(1, HID) for c in convs]
    qs = [c['q'].reshape(1, HID) for c in convs]
    r8 = jnp.arange(HEADS)[:, None]
    sel8 = (jnp.arange(HEADS * 128)[None, :] // 128 == r8).astype(F32)

    c = convs[0]
    o1 = pl.pallas_call(
        functools.partial(_k1a_body, bb),
        grid=(nb // bb,),
        in_specs=[_perg(bb, (N1, IN_C))] * 2 + [_perg(bb, (DEG, N1))] * 4
                 + [_full((HEADS, 1024))]
                 + [_full((IN_C, HID))] * 2 + [_full((HID, HEADS))] * 8
                 + [_full((HID, HID)), _full((1, HID)), _full((1, HID))],
        out_specs=[_perg(bb, (N1, HID))] * 4 + [_full((8, 128))],
        out_shape=[f32((nb, N1, HID))] * 4 + [f32((8, 128))],
        compiler_params=_cparams(),
    )(xb, xd, *srcs, sel8, c['W']['bold'], c['W']['dti'],
      *att[0][0], *att[0][1], kWs[0], kbs[0], qs[0])
    w1 = _w_pack(o1[4], nb, N1)

    def stage(l, n, p_dim, w, o_prev, a_args, a_specs, rebuild):
        cn = convs[l]
        body = functools.partial(_stage_body, bs, n, p_dim, rebuild)
        return pl.pallas_call(
            body,
            grid=(nb // bs,),
            in_specs=[_perg(bs, (n, HID))] * 4 + a_specs + [_full((8, 128))]
                     + [_full((HEADS, 1024))]
                     + [_full((HID, p_dim))] * 2 + [_full((HID, HID))] * 2
                     + [_full((HID, HEADS))] * 8
                     + [_full((HID, HID)), _full((1, HID)), _full((1, HID))],
            out_specs=[_perg(bs, (n, p_dim))] * 2 + [_perg(bs, (1, 4 * HID))]
                      + [_perg(bs, (p_dim, p_dim))] * 4
                      + [_perg(bs, (p_dim, HID))] * 4 + [_full((8, 128))],
            out_shape=[f32((nb, n, p_dim))] * 2 + [f32((nb, 1, 4 * HID))]
                      + [f32((nb, p_dim, p_dim))] * 4
                      + [f32((nb, p_dim, HID))] * 4 + [f32((8, 128))],
            compiler_params=_cparams(),
        )(*o_prev, *a_args, w, sel8,
          pools[l - 1]['Wp']['bold'], pools[l - 1]['Wp']['dti'],
          cn['W']['bold'], cn['W']['dti'],
          *att[l][0], *att[l][1], kWs[l], kbs[l], qs[l])

    r1 = stage(1, N1, P1, w1, o1[0:4], srcs, [_perg(bs, (DEG, N1))] * 4, True)
    Pb1, Pd1, x1 = r1[0], r1[1], r1[2]
    An1, o2 = r1[3:7], r1[7:11]
    w2 = _w_pack(r1[11], nb, P1)

    r2 = stage(2, P1, P2, w2, o2, An1, [_perg(bs, (P1, P1))] * 4, False)
    Pb2, Pd2, x2 = r2[0], r2[1], r2[2]
    o3 = r2[7:11]
    w3 = _w_pack(r2[11], nb, P2)

    r3 = pl.pallas_call(
        functools.partial(_k3b_body, bs),
        grid=(nb // bs,),
        in_specs=[_perg(bs, (P2, HID))] * 4 + [_full((8, 128))]
                 + [_full((HID, P3))] * 2,
        out_specs=[_perg(bs, (P2, P3))] * 2 + [_perg(bs, (1, 4 * HID))],
        out_shape=[f32((nb, P2, P3))] * 2 + [f32((nb, 1, 4 * HID))],
        compiler_params=_cparams(),
    )(*o3, w3, pools[2]['Wp']['bold'], pools[2]['Wp']['dti'])
    Pb3, Pd3, x3 = r3

    out, h = pl.pallas_call(
        _k4_body,
        grid=(1,),
        in_specs=[_full((nb, 1, 4 * HID))] * 3
                 + [_full((4 * HID, HID)), _full((1, HID)),
                    _full((HID, HID // 2)), _full((1, HID // 2)),
                    _full((HID // 2, OUT_C)), _full((1, OUT_C))],
        out_specs=[_full((nb, OUT_C)), _full((nb, HID // 2))],
        out_shape=[f32((nb, OUT_C)), f32((nb, HID // 2))],
        compiler_params=_cparams(),
    )(x1, x2, x3,
      params['lin1_W'], params['lin1_b'].reshape(1, HID),
      params['lin2_W'], params['lin2_b'].reshape(1, HID // 2),
      params['lin3_W'], params['lin3_b'].reshape(1, OUT_C))

    return (out, h, Pb1, Pd1, Pb2, Pd2, Pb3, Pd3)


# speedup vs baseline: 1.1290x; 1.0186x over previous
import functools

import jax
import jax.numpy as jnp
from jax.experimental import pallas as pl
from jax.experimental.pallas import tpu as pltpu

B_, N1, IN_C, HID, HEADS, OUT_C, DEG = 256, 90, 90, 128, 8, 2, 16
D = HID // HEADS
P1, P2, P3 = 72, 57, 45
F32 = jnp.float32
ETS = [('bb', 'b', 'b'), ('dd', 'd', 'd'), ('bd', 'b', 'd'), ('db', 'd', 'b')]
SCORE_ETS = ('bb', 'db', 'dd', 'bd')


def _dot(a, b):
    return jnp.dot(a, b, preferred_element_type=F32)


def _softmax_rows(x):
    m = jnp.max(x, axis=-1, keepdims=True)
    e = jnp.exp(x - m)
    return e / jnp.sum(e, axis=-1, keepdims=True)


def _build_A(srcT, n):
    nw = (n + 31) // 32
    val = jnp.left_shift(jnp.int32(1), jnp.bitwise_and(srcT, 31))
    word = jnp.right_shift(srcT, 5)
    rows = []
    for w in range(nw):
        m = jnp.where(word == w, val, 0)
        r = m[0:8] | m[8:16]
        r = r[0:4] | r[4:8]
        r = r[0:2] | r[2:4]
        rows.append(r[0:1] | r[1:2])
    W = jnp.concatenate(rows, axis=0).T
    bits = jax.lax.broadcasted_iota(jnp.int32, (n, 32), 1)
    segs = [jnp.bitwise_and(jnp.right_shift(W[:, w:w + 1], bits), 1)
            for w in range(nw)]
    return jnp.concatenate(segs, axis=1)[:, :n].astype(F32)


def _dot3(a, b):
    return jnp.dot(a, b, preferred_element_type=F32,
                   precision=jax.lax.Precision.HIGHEST)


def _attention_et(hs, hd, A, sel8, sel8T, sel16, asrcT, adstT):
    n = hs.shape[0]
    a_s = _dot(hs, asrcT)
    a_d = _dot(hd, adstT)
    c = a_d + jnp.max(a_s, axis=0, keepdims=True)
    c = jnp.maximum(c, 0.2 * c)
    a_sP = jnp.concatenate([a_s.T, jnp.zeros((HEADS, 128 - n), F32)], axis=1)
    y = sel8 * jnp.concatenate([a_sP] * HEADS, axis=1)
    lhs = jnp.concatenate([a_d, jnp.ones((n, HEADS), F32)], axis=1)
    u = _dot(lhs, jnp.concatenate([sel8, y], axis=0))
    cb = _dot(c, sel8)
    a128 = jnp.concatenate([A + 1e-9, jnp.zeros((n, 128 - n), F32)], axis=1)
    e_all = jnp.exp(jnp.maximum(u, 0.2 * u) - cb) \
        * jnp.concatenate([a128] * HEADS, axis=1)
    hs_pad = jnp.concatenate([hs, jnp.zeros((128 - n, HID), F32)], axis=0)
    lanes = jax.lax.broadcasted_iota(jnp.int32, (1, HID), 1)
    hp2 = jnp.concatenate(
        [jnp.where((lanes >= hh * D) & (lanes < (hh + 1) * D), hs_pad, 0.0)
         for hh in range(HEADS)], axis=0)
    onum = _dot(e_all, hp2)
    z8 = _dot(e_all, sel8T)
    zrep = _dot3(1.0 / z8, sel16)
    return jnp.maximum(onum * zrep, 0.0)


def _conv_a(bb, xb3, xd3, Amap, sels, Wb, Wd, asrcT, adstT, kW, kb, q,
            o_refs, sc_ref, n):
    h2 = {'b': _dot(xb3.reshape(bb * n, -1), Wb),
          'd': _dot(xd3.reshape(bb * n, -1), Wd)}
    h = {nt: [h2[nt][g * n:(g + 1) * n] for g in range(bb)] for nt in h2}
    o = {}
    for i, (et, s, d) in enumerate(ETS):
        o[et] = [_attention_et(h[s][g], h[d][g], Amap[et][g], *sels,
                               asrcT[et], adstT[et]) for g in range(bb)]
        for g in range(bb):
            o_refs[i][g] = o[et][g]
    big = jnp.concatenate([o[et][g] for et in SCORE_ETS for g in range(bb)],
                          axis=0)
    t = jnp.tanh(_dot(big, kW) + kb) * q
    sums = jnp.sum(t.reshape(4, bb * n, HID), axis=1)
    contrib = jnp.concatenate([sums, jnp.zeros((4, HID), F32)], axis=0)
    b = pl.program_id(0)
    prev = jnp.where(b == 0, jnp.zeros((8, 128), F32), sc_ref[...])
    sc_ref[...] = prev + contrib
    return o


def _combine3(w_ref, o_bb, o_dd, o_bd, o_db):
    xb = jnp.maximum(w_ref[0:1, 0:1] * o_bb[...] + w_ref[1:2, 0:1] * o_db[...], 0.0)
    xd = jnp.maximum(w_ref[2:3, 0:1] * o_dd[...] + w_ref[3:4, 0:1] * o_bd[...], 0.0)
    return xb, xd


def _readout(xnb, xnd):
    parts = [jnp.max(xnb, axis=0, keepdims=True),
             jnp.mean(xnb, axis=0, keepdims=True),
             jnp.max(xnd, axis=0, keepdims=True),
             jnp.mean(xnd, axis=0, keepdims=True)]
    return jnp.concatenate(parts, axis=1)


def _pool(bb, n, p_dim, xb3, xd3, Wp_b, Wp_d, A, Pb_o, Pd_o, xr_o):
    Sb2 = _softmax_rows(_dot(xb3.reshape(bb * n, HID), Wp_b))
    Sd2 = _softmax_rows(_dot(xd3.reshape(bb * n, HID), Wp_d))
    Pb_o[...] = Sb2.reshape(bb, n, p_dim)
    Pd_o[...] = Sd2.reshape(bb, n, p_dim)
    S = {'b': [Sb2[g * n:(g + 1) * n] for g in range(bb)],
         'd': [Sd2[g * n:(g + 1) * n] for g in range(bb)]}
    xnb, xnd, An = [], [], {et: [] for et, _, _ in ETS}
    for g in range(bb):
        St = {'b': S['b'][g].T, 'd': S['d'][g].T}
        xnb.append(_dot(St['b'], xb3[g]))
        xnd.append(_dot(St['d'], xd3[g]))
        xr_o[g] = _readout(xnb[-1], xnd[-1])
        for et, s, d in ETS:
            An[et].append(_dot(St[d], _dot(A[et][g], S[s][g])))
    return xnb, xnd, An


def _k1a_body(bb,
              xb_ref, xd_ref, s_bb, s_dd, s_bd, s_db, sel8, sel8T, sel16,
              Wb, Wd, as_bb, as_dd, as_bd, as_db, ad_bb, ad_dd, ad_bd, ad_db,
              kW, kb, q,
              o_bb, o_dd, o_bd, o_db, sc_ref):
    b = pl.program_id(0)
    src = {'bb': s_bb, 'dd': s_dd, 'bd': s_bd, 'db': s_db}
    Amap = {et: [_build_A(src[et][g] - (b * bb + g) * N1, N1)
                 for g in range(bb)] for et, _, _ in ETS}
    asrcT = {'bb': as_bb[...], 'dd': as_dd[...], 'bd': as_bd[...], 'db': as_db[...]}
    adstT = {'bb': ad_bb[...], 'dd': ad_dd[...], 'bd': ad_bd[...], 'db': ad_db[...]}
    _conv_a(bb, xb_ref[...], xd_ref[...], Amap,
            (sel8[...], sel8T[...], sel16[...]), Wb[...], Wd[...],
            asrcT, adstT,
            kW[...], kb[...], q[...], [o_bb, o_dd, o_bd, o_db], sc_ref, N1)


def _stage_body(bb, n, p_dim, rebuild_A,
                o1_bb, o1_dd, o1_bd, o1_db, a_bb, a_dd, a_bd, a_db, w_ref,
                sel8, sel8T, sel16, Wp_b, Wp_d, Wb, Wd,
                as_bb, as_dd, as_bd, as_db, ad_bb, ad_dd, ad_bd, ad_db,
                kW, kb, q,
                Pb_o, Pd_o, xr_o, An_bb, An_dd, An_bd, An_db,
                o2_bb, o2_dd, o2_bd, o2_db, sc_ref):
    b = pl.program_id(0)
    a_in = {'bb': a_bb, 'dd': a_dd, 'bd': a_bd, 'db': a_db}
    if rebuild_A:
        A = {et: [_build_A(a_in[et][g] - (b * bb + g) * n, n)
                  for g in range(bb)] for et, _, _ in ETS}
    else:
        A = {et: [a_in[et][g] for g in range(bb)] for et, _, _ in ETS}
    xb3, xd3 = _combine3(w_ref, o1_bb, o1_dd, o1_bd, o1_db)
    xnb, xnd, An = _pool(bb, n, p_dim, xb3, xd3, Wp_b[...], Wp_d[...], A,
                         Pb_o, Pd_o, xr_o)
    An_o = {'bb': An_bb, 'dd': An_dd, 'bd': An_bd, 'db': An_db}
    for et, _, _ in ETS:
        for g in range(bb):
            An_o[et][g] = An[et][g]
    asrcT = {'bb': as_bb[...], 'dd': as_dd[...], 'bd': as_bd[...], 'db': as_db[...]}
    adstT = {'bb': ad_bb[...], 'dd': ad_dd[...], 'bd': ad_bd[...], 'db': ad_db[...]}
    xnb3 = jnp.concatenate([x[None] for x in xnb], axis=0)
    xnd3 = jnp.concatenate([x[None] for x in xnd], axis=0)
    _conv_a(bb, xnb3, xnd3, An, (sel8[...], sel8T[...], sel16[...]),
            Wb[...], Wd[...], asrcT, adstT,
            kW[...], kb[...], q[...], [o2_bb, o2_dd, o2_bd, o2_db], sc_ref,
            p_dim)


def _k3b_body(bb, o_bb, o_dd, o_bd, o_db, w_ref, Wp_b, Wp_d, Pb_o, Pd_o, xr_o):
    n, p_dim = P2, P3
    xb3, xd3 = _combine3(w_ref, o_bb, o_dd, o_bd, o_db)
    Sb2 = _softmax_rows(_dot(xb3.reshape(bb * n, HID), Wp_b[...]))
    Sd2 = _softmax_rows(_dot(xd3.reshape(bb * n, HID), Wp_d[...]))
    Pb_o[...] = Sb2.reshape(bb, n, p_dim)
    Pd_o[...] = Sd2.reshape(bb, n, p_dim)
    for g in range(bb):
        xnb = _dot(Sb2[g * n:(g + 1) * n].T, xb3[g])
        xnd = _dot(Sd2[g * n:(g + 1) * n].T, xd3[g])
        xr_o[g] = _readout(xnb, xnd)


def _k4_body(x1_ref, x2_ref, x3_ref, l1W, l1b, l2W, l2b, l3W, l3b,
             out_o, h_o):
    nb = x1_ref.shape[0]
    s = (x1_ref[...] + x2_ref[...] + x3_ref[...]).reshape(nb, 4 * HID)
    s = s - jnp.mean(s, axis=0, keepdims=True)
    rn = jnp.sqrt(1e-6 + jnp.sum(s * s, axis=1, keepdims=True))
    feat = 100.0 * s / rn
    h1 = jnp.maximum(_dot(feat, l1W[...]) + l1b[...], 0.0)
    h2 = jnp.maximum(_dot(h1, l2W[...]) + l2b[...], 0.0)
    out_o[...] = _dot(h2, l3W[...]) + l3b[...]
    h_o[...] = h2


def _att_mat(a):
    m = jnp.zeros((HID, HEADS), F32)
    return m.at[jnp.arange(HID), jnp.arange(HID) // D].set(a.reshape(HID))


def _w_pack(score_out, nb, n):
    s = jnp.sum(score_out, axis=1)[0:4] / (nb * n)
    w = jnp.concatenate([jax.nn.softmax(s[0:2]), jax.nn.softmax(s[2:4])])
    w = jnp.concatenate([w, jnp.zeros((4,), F32)])
    return jnp.broadcast_to(w[:, None], (8, 128))


def _full(shape):
    nd = len(shape)
    return pl.BlockSpec(shape, lambda b, _nd=nd: (0,) * _nd)


def _perg(bb, shape):
    nd = len(shape)
    return pl.BlockSpec((bb,) + shape, lambda b, _nd=nd: (b,) + (0,) * _nd)


def _cparams():
    return pltpu.CompilerParams(dimension_semantics=("arbitrary",))


def kernel(x_bold, x_dti, ei_bb, ei_dd, ei_bd, ei_db, params):
    nb = x_bold.shape[0] // N1
    bb = 4 if nb % 4 == 0 else 1
    bs = 8 if nb % 8 == 0 else bb
    f32 = lambda shape: jax.ShapeDtypeStruct(shape, F32)

    srcs = [ei[0].astype(jnp.int32).reshape(nb, N1, DEG).transpose(0, 2, 1)
            for ei in (ei_bb, ei_dd, ei_bd, ei_db)]
    xb = x_bold.reshape(nb, N1, IN_C)
    xd = x_dti.reshape(nb, N1, IN_C)

    convs = [params['conv%d' % (l + 1)] for l in range(3)]
    pools = [params['pool%d' % (l + 1)] for l in range(3)]
    att = []
    for c in convs:
        att.append(([_att_mat(c['att_src'][et]) for et, _, _ in ETS],
                    [_att_mat(c['att_dst'][et]) for et, _, _ in ETS]))
    kWs = [c['k_W'] for c in convs]
    kbs = [c['k_b'].reshape(1, HID) for c in convs]
    qs = [c['q'].reshape(1, HID) for c in convs]
    r8 = jnp.arange(HEADS)[:, None]
    sel8 = (jnp.arange(HEADS * 128)[None, :] // 128 == r8).astype(F32)
    sel8T = sel8.T
    sel16 = (jnp.arange(HID)[None, :] // D == r8).astype(F32)

    c = convs[0]
    o1 = pl.pallas_call(
        functools.partial(_k1a_body, bb),
        grid=(nb // bb,),
        in_specs=[_perg(bb, (N1, IN_C))] * 2 + [_perg(bb, (DEG, N1))] * 4
                 + [_full((HEADS, 1024)), _full((1024, HEADS)),
                    _full((HEADS, HID))]
                 + [_full((IN_C, HID))] * 2 + [_full((HID, HEADS))] * 8
                 + [_full((HID, HID)), _full((1, HID)), _full((1, HID))],
        out_specs=[_perg(bb, (N1, HID))] * 4 + [_full((8, 128))],
        out_shape=[f32((nb, N1, HID))] * 4 + [f32((8, 128))],
        compiler_params=_cparams(),
    )(xb, xd, *srcs, sel8, sel8T, sel16, c['W']['bold'], c['W']['dti'],
      *att[0][0], *att[0][1], kWs[0], kbs[0], qs[0])
    w1 = _w_pack(o1[4], nb, N1)

    def stage(l, n, p_dim, w, o_prev, a_args, a_specs, rebuild):
        cn = convs[l]
        body = functools.partial(_stage_body, bs, n, p_dim, rebuild)
        return pl.pallas_call(
            body,
            grid=(nb // bs,),
            in_specs=[_perg(bs, (n, HID))] * 4 + a_specs + [_full((8, 128))]
                     + [_full((HEADS, 1024)), _full((1024, HEADS)),
                    _full((HEADS, HID))]
                     + [_full((HID, p_dim))] * 2 + [_full((HID, HID))] * 2
                     + [_full((HID, HEADS))] * 8
                     + [_full((HID, HID)), _full((1, HID)), _full((1, HID))],
            out_specs=[_perg(bs, (n, p_dim))] * 2 + [_perg(bs, (1, 4 * HID))]
                      + [_perg(bs, (p_dim, p_dim))] * 4
                      + [_perg(bs, (p_dim, HID))] * 4 + [_full((8, 128))],
            out_shape=[f32((nb, n, p_dim))] * 2 + [f32((nb, 1, 4 * HID))]
                      + [f32((nb, p_dim, p_dim))] * 4
                      + [f32((nb, p_dim, HID))] * 4 + [f32((8, 128))],
            compiler_params=_cparams(),
        )(*o_prev, *a_args, w, sel8, sel8T, sel16,
          pools[l - 1]['Wp']['bold'], pools[l - 1]['Wp']['dti'],
          cn['W']['bold'], cn['W']['dti'],
          *att[l][0], *att[l][1], kWs[l], kbs[l], qs[l])

    r1 = stage(1, N1, P1, w1, o1[0:4], srcs, [_perg(bs, (DEG, N1))] * 4, True)
    Pb1, Pd1, x1 = r1[0], r1[1], r1[2]
    An1, o2 = r1[3:7], r1[7:11]
    w2 = _w_pack(r1[11], nb, P1)

    r2 = stage(2, P1, P2, w2, o2, An1, [_perg(bs, (P1, P1))] * 4, False)
    Pb2, Pd2, x2 = r2[0], r2[1], r2[2]
    o3 = r2[7:11]
    w3 = _w_pack(r2[11], nb, P2)

    r3 = pl.pallas_call(
        functools.partial(_k3b_body, bs),
        grid=(nb // bs,),
        in_specs=[_perg(bs, (P2, HID))] * 4 + [_full((8, 128))]
                 + [_full((HID, P3))] * 2,
        out_specs=[_perg(bs, (P2, P3))] * 2 + [_perg(bs, (1, 4 * HID))],
        out_shape=[f32((nb, P2, P3))] * 2 + [f32((nb, 1, 4 * HID))],
        compiler_params=_cparams(),
    )(*o3, w3, pools[2]['Wp']['bold'], pools[2]['Wp']['dti'])
    Pb3, Pd3, x3 = r3

    out, h = pl.pallas_call(
        _k4_body,
        grid=(1,),
        in_specs=[_full((nb, 1, 4 * HID))] * 3
                 + [_full((4 * HID, HID)), _full((1, HID)),
                    _full((HID, HID // 2)), _full((1, HID // 2)),
                    _full((HID // 2, OUT_C)), _full((1, OUT_C))],
        out_specs=[_full((nb, OUT_C)), _full((nb, HID // 2))],
        out_shape=[f32((nb, OUT_C)), f32((nb, HID // 2))],
        compiler_params=_cparams(),
    )(x1, x2, x3,
      params['lin1_W'], params['lin1_b'].reshape(1, HID),
      params['lin2_W'], params['lin2_b'].reshape(1, HID // 2),
      params['lin3_W'], params['lin3_b'].reshape(1, OUT_C))

    return (out, h, Pb1, Pd1, Pb2, Pd2, Pb3, Pd3)
